# baseline (speedup 1.0000x reference)
import jax
import jax.numpy as jnp
from jax.experimental import pallas as pl
from jax.experimental.pallas import tpu as pltpu

_PAD = 1
_SBLK = 256


def _body(tokT_ref, pad_ref, wa_ref, wb_ref, out_ref):
    j = pl.program_id(0)
    w = jnp.concatenate(
        [wa_ref[pl.ds(2, _SBLK - 2), :], wb_ref[pl.ds(0, 2), :]], axis=0)
    pad = pad_ref[...]
    bsz = tokT_ref.shape[1]
    for b in range(bsz):
        mask = tokT_ref[pl.ds(j * _SBLK, _SBLK), pl.ds(b, 1)] != _PAD
        out_ref[b, :, :] = jnp.where(mask, w, pad)


def kernel(input, weights):
    bsz, seq_len = input.shape
    dim = weights.shape[1]
    pad_row = jax.lax.slice(weights, (_PAD, 0), (_PAD + 1, dim))
    tokT = input.T
    grid = (seq_len // _SBLK,)
    out = pl.pallas_call(
        _body,
        grid=grid,
        in_specs=[
            pl.BlockSpec((seq_len, bsz), lambda j: (0, 0)),
            pl.BlockSpec((1, dim), lambda j: (0, 0)),
            pl.BlockSpec((_SBLK, dim), lambda j: (j, 0)),
            pl.BlockSpec((8, dim), lambda j: ((j + 1) * (_SBLK // 8), 0)),
        ],
        out_specs=pl.BlockSpec((bsz, _SBLK, dim), lambda j: (0, j, 0)),
        out_shape=jax.ShapeDtypeStruct((bsz, seq_len, dim), jnp.float32),
    )(tokT, pad_row, weights, weights)
    return out
